# baseline (speedup 1.0000x reference)
import functools

import jax
import jax.numpy as jnp
from jax import lax
from jax.experimental import pallas as pl
from jax.experimental.pallas import tpu as pltpu
from jax.experimental.pallas import tpu_sc as plsc

D_IN = 768
NUM_LATENTS = 24576
K = 32
N_TOK = 2048

TM = 256
TN = 1024
CHUNK_SIZES = (1024, 1024)


def _encode_body(x_ref, w_ref, benc_ref, bdec_ref, out_ref):
    xc = x_ref[...] - bdec_ref[...]
    acts = lax.dot_general(
        xc, w_ref[...], (((1,), (1,)), ((), ())),
        preferred_element_type=jnp.float32,
    )
    out_ref[...] = jnp.maximum(acts + benc_ref[...], 0.0)


def _encode(x, W_enc, b_enc, b_dec):
    n = x.shape[0]
    grid = (n // TM, NUM_LATENTS // TN)
    return pl.pallas_call(
        _encode_body,
        grid=grid,
        in_specs=[
            pl.BlockSpec((TM, D_IN), lambda i, j: (i, 0)),
            pl.BlockSpec((TN, D_IN), lambda i, j: (j, 0)),
            pl.BlockSpec((1, TN), lambda i, j: (0, j)),
            pl.BlockSpec((1, D_IN), lambda i, j: (0, 0)),
        ],
        out_specs=pl.BlockSpec((TM, TN), lambda i, j: (i, j)),
        out_shape=jax.ShapeDtypeStruct((n, NUM_LATENTS), jnp.float32),
        compiler_params=pltpu.CompilerParams(
            dimension_semantics=("parallel", "parallel"),
        ),
    )(x, W_enc, b_enc.reshape(1, NUM_LATENTS), b_dec.reshape(1, D_IN))


NC = 2
NS = 16
NW = NC * NS
TPW = N_TOK // NW
L = 16
NVR = NUM_LATENTS // L
NG = NVR // L
NH = NG // L
DCH = D_IN // L
NEG = -1.0


def _make_sc_body(tpw):
  def _sc_body(pre_hbm, x_hbm, wdec_hbm, bdec_hbm,
               sae_hbm, ta_hbm, ti_hbm, pcol_hbm, pse_hbm,
               row2, x4, out2, wrows2, l1v, l1i, l2v, l2i, rv, ri, tik2,
               tva, tia, bdec_v, colacc, s16f,
               sem_row, sem_x, sem_out, sem_g):
    wid = lax.axis_index("s") * NC + lax.axis_index("c")
    w_base = wid * tpw
    iota = lax.iota(jnp.int32, L)
    zf = jnp.zeros((L,), jnp.float32)
    zi = jnp.zeros((L,), jnp.int32)
    negv = jnp.full((L,), NEG, jnp.float32)
    jconsts = [jnp.full((L,), j, jnp.int32) for j in range(L)]

    pltpu.sync_copy(bdec_hbm, bdec_v)
    pltpu.async_copy(pre_hbm.at[w_base], row2.at[0], sem_row)
    pltpu.async_copy(x_hbm.at[w_base], x4.at[pl.ds(0, D_IN)], sem_x)
    for d in range(DCH):
        colacc[pl.ds(d * L, L)] = zf

    def _tree(pairs):
        while len(pairs) > 1:
            nxt = []
            for a in range(0, len(pairs) - 1, 2):
                (av, ai), (bv, bi) = pairs[a], pairs[a + 1]
                m = bv > av
                nxt.append((jnp.where(m, bv, av), jnp.where(m, bi, ai)))
            if len(pairs) % 2:
                nxt.append(pairs[-1])
            pairs = nxt
        return pairs[0]

    def fold_l1(row_ref, g_el):
        base = g_el * (L * L)
        pairs = [(row_ref[pl.ds(base + j * L, L)], jconsts[j])
                 for j in range(L)]
        acc_v, acc_j = _tree(pairs)
        l1v[pl.ds(g_el * L, L)] = acc_v
        l1i[pl.ds(g_el * L, L)] = acc_j

    def fold_l2(h_el):
        base = h_el * (L * L)
        pairs = [(l1v[pl.ds(base + g * L, L)],
                  l1i[pl.ds(base + g * L, L)] + jconsts[g] * L)
                 for g in range(L)]
        acc_v, acc_i = _tree(pairs)
        l2v[pl.ds(h_el * L, L)] = acc_v
        l2i[pl.ds(h_el * L, L)] = acc_i

    def fold_r():
        pairs = [(l2v[pl.ds(h * L, L)],
                  l2i[pl.ds(h * L, L)] + jconsts[h] * (L * L))
                 for h in range(NH)]
        return _tree(pairs)

    def decode_token(q, phq, pv0, pv1, sqv, e2v):
        q_glob = w_base + q
        pltpu.make_async_copy(
            wdec_hbm.at[tik2.at[phq]], wrows2.at[phq], sem_g).wait()
        avs = ([jnp.broadcast_to(pv0[k], (L,)) for k in range(L)]
               + [jnp.broadcast_to(pv1[k], (L,)) for k in range(L)])
        xbase = (q & 3) * D_IN

        def dchunk(d, car2):
            sqv_, e2v_ = car2
            sl = pl.ds(d * L, L)
            terms = [avs[k] * wrows2[phq, k, sl] for k in range(K)]
            terms.append(bdec_v[sl])
            while len(terms) > 1:
                terms = ([terms[a] + terms[a + 1]
                          for a in range(0, len(terms) - 1, 2)]
                         + ([terms[-1]] if len(terms) % 2 else []))
            acc = terms[0]
            xv = x4[pl.ds(xbase + d * L, L)]
            out2[phq, sl] = acc
            e = acc - xv
            colacc[sl] = colacc[sl] + xv
            return sqv_ + xv * xv, e2v_ + e * e

        sqv, e2v = lax.fori_loop(0, DCH, dchunk, (sqv, e2v))
        pltpu.async_copy(out2.at[phq], sae_hbm.at[q_glob], sem_out)
        return sqv, e2v

    def token_step(tt, ph, carry, do_decode=True):
        sqv, e2v, pv0, pv1 = carry
        tt = jnp.int32(tt)
        t_glob = w_base + tt
        pltpu.make_async_copy(pre_hbm.at[t_glob], row2.at[ph], sem_row).wait()
        pltpu.make_async_copy(
            x_hbm.at[t_glob],
            x4.at[pl.ds((tt & 3) * D_IN, D_IN)], sem_x).wait()

        @pl.when(tt + 1 < tpw)
        def _prefetch():
            pltpu.async_copy(pre_hbm.at[t_glob + 1], row2.at[1 - ph], sem_row)
            pltpu.async_copy(
                x_hbm.at[t_glob + 1],
                x4.at[pl.ds(((tt + 1) & 3) * D_IN, D_IN)], sem_x)

        row_ref = row2.at[ph]

        def build1(g, _):
            fold_l1(row_ref, g)
            return 0
        lax.fori_loop(0, NG, build1, 0)
        for h in range(NH):
            fold_l2(h)
        rv0, ri0 = fold_r()

        def extract(i, car):
            ov0, oi0, ov1, oi1, rvr, rir = car
            bv = rvr
            bg = rir * L + iota
            for dd in (8, 4, 2, 1):
                perm = iota ^ dd
                o_v = jnp.take(bv, perm)
                o_g = jnp.take(bg, perm)
                better = jnp.logical_or(
                    o_v > bv, jnp.logical_and(o_v == bv, o_g < bg))
                bv = jnp.where(better, o_v, bv)
                bg = jnp.where(better, o_g, bg)
            m0 = iota == jnp.broadcast_to(i, (L,))
            m1 = iota == jnp.broadcast_to(i - L, (L,))
            ov0 = jnp.where(m0, bv, ov0)
            oi0 = jnp.where(m0, bg, oi0)
            ov1 = jnp.where(m1, bv, ov1)
            oi1 = jnp.where(m1, bg, oi1)
            gidx = bg[0]
            vj = gidx >> 4
            lane = gidx & (L - 1)
            off = vj * L
            vcur = row_ref[pl.ds(off, L)]
            row_ref[pl.ds(off, L)] = jnp.where(
                iota == jnp.broadcast_to(lane, (L,)), negv, vcur)
            fold_l1(row_ref, vj >> 4)
            fold_l2(vj >> 8)
            rvr, rir = fold_r()
            return ov0, oi0, ov1, oi1, rvr, rir

        ov0, oi0, ov1, oi1, _, _ = lax.fori_loop(
            0, K, extract, (negv, zi, negv, zi, rv0, ri0))

        off_t = tt * K
        tik2[ph, pl.ds(0, L)] = oi0
        tik2[ph, pl.ds(L, L)] = oi1
        tva[pl.ds(off_t, L)] = ov0
        tva[pl.ds(off_t + L, L)] = ov1
        tia[pl.ds(off_t, L)] = oi0
        tia[pl.ds(off_t + L, L)] = oi1

        pltpu.async_copy(wdec_hbm.at[tik2.at[ph]], wrows2.at[ph], sem_g)

        if do_decode:
            @pl.when(tt >= 3)
            def _drain_out():
                pltpu.make_async_copy(
                    out2.at[1 - ph], sae_hbm.at[t_glob - 3], sem_out).wait()
            sqv, e2v = decode_token(tt - 1, 1 - ph, pv0, pv1, sqv, e2v)
        return sqv, e2v, ov0, ov1

    carry = token_step(0, 0, (zf, zf, zf, zf), do_decode=False)

    def pair_step(p, carry):
        carry = token_step(2 * p + 1, 1, carry)
        carry = token_step(2 * p + 2, 0, carry)
        return carry

    carry = lax.fori_loop(0, (tpw - 2) // 2, pair_step, carry)
    sqv, e2v, pv0, pv1 = token_step(tpw - 1, 1, carry)

    phl = (tpw - 1) & 1
    pltpu.make_async_copy(
        out2.at[phl], sae_hbm.at[w_base + tpw - 3], sem_out).wait()
    sqv, e2v = decode_token(tpw - 1, phl, pv0, pv1, sqv, e2v)
    pltpu.make_async_copy(
        out2.at[1 - phl], sae_hbm.at[w_base + tpw - 2], sem_out).wait()
    pltpu.make_async_copy(
        out2.at[phl], sae_hbm.at[w_base + tpw - 1], sem_out).wait()

    pltpu.sync_copy(tva, ta_hbm.at[pl.ds(w_base * K, tpw * K)])
    pltpu.sync_copy(tia, ti_hbm.at[pl.ds(w_base * K, tpw * K)])

    s16f[...] = sqv
    pltpu.sync_copy(s16f, pse_hbm.at[pl.ds(wid * 2 * L, L)])
    s16f[...] = e2v
    pltpu.sync_copy(s16f, pse_hbm.at[pl.ds(wid * 2 * L + L, L)])
    pltpu.sync_copy(colacc, pcol_hbm.at[pl.ds(wid * D_IN, D_IN)])

  return _sc_body


def _sc_decode(pre_acts, x, W_dec, b_dec):
    n = pre_acts.shape[0]
    tpw = n // NW
    mesh = plsc.VectorSubcoreMesh(
        core_axis_name="c", subcore_axis_name="s",
        num_cores=NC, num_subcores=NS)
    out_type = (
        jax.ShapeDtypeStruct((n, D_IN), jnp.float32),
        jax.ShapeDtypeStruct((n * K,), jnp.float32),
        jax.ShapeDtypeStruct((n * K,), jnp.int32),
        jax.ShapeDtypeStruct((NW * D_IN,), jnp.float32),
        jax.ShapeDtypeStruct((NW * 2 * L,), jnp.float32),
    )
    scratch = [
        pltpu.VMEM((2, NUM_LATENTS), jnp.float32),
        pltpu.VMEM((4 * D_IN,), jnp.float32),
        pltpu.VMEM((2, D_IN), jnp.float32),
        pltpu.VMEM((2, K, D_IN), jnp.float32),
        pltpu.VMEM((NG * L,), jnp.float32),
        pltpu.VMEM((NG * L,), jnp.int32),
        pltpu.VMEM((NH * L,), jnp.float32),
        pltpu.VMEM((NH * L,), jnp.int32),
        pltpu.VMEM((L,), jnp.float32),
        pltpu.VMEM((L,), jnp.int32),
        pltpu.VMEM((2, K), jnp.int32),
        pltpu.VMEM((tpw * K,), jnp.float32),
        pltpu.VMEM((tpw * K,), jnp.int32),
        pltpu.VMEM((D_IN,), jnp.float32),
        pltpu.VMEM((D_IN,), jnp.float32),
        pltpu.VMEM((L,), jnp.float32),
        pltpu.SemaphoreType.DMA,
        pltpu.SemaphoreType.DMA,
        pltpu.SemaphoreType.DMA,
        pltpu.SemaphoreType.DMA,
    ]
    return pl.kernel(
        _make_sc_body(tpw), out_type=out_type, mesh=mesh,
        scratch_types=scratch,
    )(pre_acts, x, W_dec, b_dec)


def kernel(x, W_enc, b_enc, W_dec, b_dec):
    outs = []
    off = 0
    for n_c in CHUNK_SIZES:
        xc = x[off:off + n_c]
        off += n_c
        pre_c = _encode(xc, W_enc, b_enc, b_dec)
        outs.append(_sc_decode(pre_c, xc, W_dec, b_dec))
    nck = len(CHUNK_SIZES)
    sae_out = jnp.concatenate([o[0] for o in outs], axis=0)
    ta = jnp.concatenate([o[1] for o in outs])
    ti = jnp.concatenate([o[2] for o in outs])
    pcol = jnp.stack([o[3] for o in outs])
    pse = jnp.stack([o[4] for o in outs])
    top_acts = ta.reshape(N_TOK, K)
    top_indices = ti.reshape(N_TOK, K)
    colsum = pcol.reshape(nck * NW, D_IN).sum(axis=0)
    pse2 = pse.reshape(nck * NW, 2, L)
    sq_tot = pse2[:, 0, :].sum()
    e2_tot = pse2[:, 1, :].sum()
    total_variance = sq_tot - jnp.sum(colsum * colsum) / N_TOK
    fvu = e2_tot / total_variance
    auxk_loss = jnp.array(0.0, dtype=jnp.float32)
    multi_topk_fvu = jnp.array(0.0, dtype=jnp.float32)
    return (sae_out, top_acts, top_indices, fvu, auxk_loss, multi_topk_fvu)


# speedup vs baseline: 1.1127x; 1.1127x over previous
import functools

import jax
import jax.numpy as jnp
from jax import lax
from jax.experimental import pallas as pl
from jax.experimental.pallas import tpu as pltpu
from jax.experimental.pallas import tpu_sc as plsc

D_IN = 768
NUM_LATENTS = 24576
K = 32
N_TOK = 2048

TM = 512
TN = 1024
CHUNK_SIZES = (512, 512, 512, 512)


def _encode_body(x_ref, w_ref, benc_ref, bdec_ref, out_ref):
    xc = x_ref[...] - bdec_ref[...]
    acts = lax.dot_general(
        xc, w_ref[...], (((1,), (1,)), ((), ())),
        preferred_element_type=jnp.float32,
    )
    out_ref[...] = jnp.maximum(acts + benc_ref[...], 0.0)


def _encode(x, W_enc, b_enc, b_dec):
    n = x.shape[0]
    grid = (n // TM, NUM_LATENTS // TN)
    return pl.pallas_call(
        _encode_body,
        grid=grid,
        in_specs=[
            pl.BlockSpec((TM, D_IN), lambda i, j: (i, 0)),
            pl.BlockSpec((TN, D_IN), lambda i, j: (j, 0)),
            pl.BlockSpec((1, TN), lambda i, j: (0, j)),
            pl.BlockSpec((1, D_IN), lambda i, j: (0, 0)),
        ],
        out_specs=pl.BlockSpec((TM, TN), lambda i, j: (i, j)),
        out_shape=jax.ShapeDtypeStruct((n, NUM_LATENTS), jnp.float32),
        compiler_params=pltpu.CompilerParams(
            dimension_semantics=("parallel", "parallel"),
        ),
    )(x, W_enc, b_enc.reshape(1, NUM_LATENTS), b_dec.reshape(1, D_IN))


NC = 2
NS = 16
NW = NC * NS
TPW = N_TOK // NW
L = 16
NVR = NUM_LATENTS // L
NG = NVR // L
NH = NG // L
DCH = D_IN // L
NEG = -1.0


def _make_sc_body(tpw):
  def _sc_body(pre_hbm, x_hbm, wdec_hbm, bdec_hbm,
               sae_hbm, ta_hbm, ti_hbm, pcol_hbm, pse_hbm,
               row2, x4, out2, wrows2, l1v, l1i, l2v, l2i, rv, ri, tik2,
               tva, tia, bdec_v, colacc, s16f,
               sem_row, sem_x, sem_out, sem_g):
    wid = lax.axis_index("s") * NC + lax.axis_index("c")
    w_base = wid * tpw
    iota = lax.iota(jnp.int32, L)
    zf = jnp.zeros((L,), jnp.float32)
    zi = jnp.zeros((L,), jnp.int32)
    negv = jnp.full((L,), NEG, jnp.float32)
    jconsts = [jnp.full((L,), j, jnp.int32) for j in range(L)]

    pltpu.sync_copy(bdec_hbm, bdec_v)
    pltpu.async_copy(pre_hbm.at[w_base], row2.at[0], sem_row)
    pltpu.async_copy(x_hbm.at[w_base], x4.at[pl.ds(0, D_IN)], sem_x)
    for d in range(DCH):
        colacc[pl.ds(d * L, L)] = zf

    def _tree(pairs):
        while len(pairs) > 1:
            nxt = []
            for a in range(0, len(pairs) - 1, 2):
                (av, ai), (bv, bi) = pairs[a], pairs[a + 1]
                m = bv > av
                nxt.append((jnp.where(m, bv, av), jnp.where(m, bi, ai)))
            if len(pairs) % 2:
                nxt.append(pairs[-1])
            pairs = nxt
        return pairs[0]

    def fold_l1(row_ref, g_el):
        base = g_el * (L * L)
        pairs = [(row_ref[pl.ds(base + j * L, L)], jconsts[j])
                 for j in range(L)]
        acc_v, acc_j = _tree(pairs)
        l1v[pl.ds(g_el * L, L)] = acc_v
        l1i[pl.ds(g_el * L, L)] = acc_j

    def fold_l2(h_el):
        base = h_el * (L * L)
        pairs = [(l1v[pl.ds(base + g * L, L)],
                  l1i[pl.ds(base + g * L, L)] + jconsts[g] * L)
                 for g in range(L)]
        acc_v, acc_i = _tree(pairs)
        l2v[pl.ds(h_el * L, L)] = acc_v
        l2i[pl.ds(h_el * L, L)] = acc_i

    def fold_r():
        pairs = [(l2v[pl.ds(h * L, L)],
                  l2i[pl.ds(h * L, L)] + jconsts[h] * (L * L))
                 for h in range(NH)]
        return _tree(pairs)

    def decode_token(q, phq, pv0, pv1, sqv, e2v):
        q_glob = w_base + q
        pltpu.make_async_copy(
            wdec_hbm.at[tik2.at[phq]], wrows2.at[phq], sem_g).wait()
        avs = ([jnp.broadcast_to(pv0[k], (L,)) for k in range(L)]
               + [jnp.broadcast_to(pv1[k], (L,)) for k in range(L)])
        xbase = (q & 3) * D_IN

        def dchunk(d, car2):
            sqv_, e2v_ = car2
            sl = pl.ds(d * L, L)
            terms = [avs[k] * wrows2[phq, k, sl] for k in range(K)]
            terms.append(bdec_v[sl])
            while len(terms) > 1:
                terms = ([terms[a] + terms[a + 1]
                          for a in range(0, len(terms) - 1, 2)]
                         + ([terms[-1]] if len(terms) % 2 else []))
            acc = terms[0]
            xv = x4[pl.ds(xbase + d * L, L)]
            out2[phq, sl] = acc
            e = acc - xv
            colacc[sl] = colacc[sl] + xv
            return sqv_ + xv * xv, e2v_ + e * e

        sqv, e2v = lax.fori_loop(0, DCH, dchunk, (sqv, e2v))
        pltpu.async_copy(out2.at[phq], sae_hbm.at[q_glob], sem_out)
        return sqv, e2v

    def token_step(tt, ph, carry, do_decode=True):
        sqv, e2v, pv0, pv1 = carry
        tt = jnp.int32(tt)
        t_glob = w_base + tt
        pltpu.make_async_copy(pre_hbm.at[t_glob], row2.at[ph], sem_row).wait()
        pltpu.make_async_copy(
            x_hbm.at[t_glob],
            x4.at[pl.ds((tt & 3) * D_IN, D_IN)], sem_x).wait()

        @pl.when(tt + 1 < tpw)
        def _prefetch():
            pltpu.async_copy(pre_hbm.at[t_glob + 1], row2.at[1 - ph], sem_row)
            pltpu.async_copy(
                x_hbm.at[t_glob + 1],
                x4.at[pl.ds(((tt + 1) & 3) * D_IN, D_IN)], sem_x)

        row_ref = row2.at[ph]

        def build1(g, _):
            fold_l1(row_ref, g)
            return 0
        lax.fori_loop(0, NG, build1, 0)
        for h in range(NH):
            fold_l2(h)
        rv0, ri0 = fold_r()

        def extract(i, car):
            ov0, oi0, ov1, oi1, rvr, rir = car
            bv = rvr
            bg = rir * L + iota
            for dd in (8, 4, 2, 1):
                perm = iota ^ dd
                o_v = jnp.take(bv, perm)
                o_g = jnp.take(bg, perm)
                better = jnp.logical_or(
                    o_v > bv, jnp.logical_and(o_v == bv, o_g < bg))
                bv = jnp.where(better, o_v, bv)
                bg = jnp.where(better, o_g, bg)
            m0 = iota == jnp.broadcast_to(i, (L,))
            m1 = iota == jnp.broadcast_to(i - L, (L,))
            ov0 = jnp.where(m0, bv, ov0)
            oi0 = jnp.where(m0, bg, oi0)
            ov1 = jnp.where(m1, bv, ov1)
            oi1 = jnp.where(m1, bg, oi1)
            gidx = bg[0]
            vj = gidx >> 4
            lane = gidx & (L - 1)
            off = vj * L
            vcur = row_ref[pl.ds(off, L)]
            row_ref[pl.ds(off, L)] = jnp.where(
                iota == jnp.broadcast_to(lane, (L,)), negv, vcur)
            fold_l1(row_ref, vj >> 4)
            fold_l2(vj >> 8)
            rvr, rir = fold_r()
            return ov0, oi0, ov1, oi1, rvr, rir

        ov0, oi0, ov1, oi1, _, _ = lax.fori_loop(
            0, K, extract, (negv, zi, negv, zi, rv0, ri0))

        off_t = tt * K
        tik2[ph, pl.ds(0, L)] = oi0
        tik2[ph, pl.ds(L, L)] = oi1
        tva[pl.ds(off_t, L)] = ov0
        tva[pl.ds(off_t + L, L)] = ov1
        tia[pl.ds(off_t, L)] = oi0
        tia[pl.ds(off_t + L, L)] = oi1

        pltpu.async_copy(wdec_hbm.at[tik2.at[ph]], wrows2.at[ph], sem_g)

        if do_decode:
            @pl.when(tt >= 3)
            def _drain_out():
                pltpu.make_async_copy(
                    out2.at[1 - ph], sae_hbm.at[t_glob - 3], sem_out).wait()
            sqv, e2v = decode_token(tt - 1, 1 - ph, pv0, pv1, sqv, e2v)
        return sqv, e2v, ov0, ov1

    carry = token_step(0, 0, (zf, zf, zf, zf), do_decode=False)

    def pair_step(p, carry):
        carry = token_step(2 * p + 1, 1, carry)
        carry = token_step(2 * p + 2, 0, carry)
        return carry

    carry = lax.fori_loop(0, (tpw - 2) // 2, pair_step, carry)
    sqv, e2v, pv0, pv1 = token_step(tpw - 1, 1, carry)

    phl = (tpw - 1) & 1
    pltpu.make_async_copy(
        out2.at[phl], sae_hbm.at[w_base + tpw - 3], sem_out).wait()
    sqv, e2v = decode_token(tpw - 1, phl, pv0, pv1, sqv, e2v)
    pltpu.make_async_copy(
        out2.at[1 - phl], sae_hbm.at[w_base + tpw - 2], sem_out).wait()
    pltpu.make_async_copy(
        out2.at[phl], sae_hbm.at[w_base + tpw - 1], sem_out).wait()

    pltpu.sync_copy(tva, ta_hbm.at[pl.ds(w_base * K, tpw * K)])
    pltpu.sync_copy(tia, ti_hbm.at[pl.ds(w_base * K, tpw * K)])

    s16f[...] = sqv
    pltpu.sync_copy(s16f, pse_hbm.at[pl.ds(wid * 2 * L, L)])
    s16f[...] = e2v
    pltpu.sync_copy(s16f, pse_hbm.at[pl.ds(wid * 2 * L + L, L)])
    pltpu.sync_copy(colacc, pcol_hbm.at[pl.ds(wid * D_IN, D_IN)])

  return _sc_body


def _sc_decode(pre_acts, x, W_dec, b_dec):
    n = pre_acts.shape[0]
    tpw = n // NW
    mesh = plsc.VectorSubcoreMesh(
        core_axis_name="c", subcore_axis_name="s",
        num_cores=NC, num_subcores=NS)
    out_type = (
        jax.ShapeDtypeStruct((n, D_IN), jnp.float32),
        jax.ShapeDtypeStruct((n * K,), jnp.float32),
        jax.ShapeDtypeStruct((n * K,), jnp.int32),
        jax.ShapeDtypeStruct((NW * D_IN,), jnp.float32),
        jax.ShapeDtypeStruct((NW * 2 * L,), jnp.float32),
    )
    scratch = [
        pltpu.VMEM((2, NUM_LATENTS), jnp.float32),
        pltpu.VMEM((4 * D_IN,), jnp.float32),
        pltpu.VMEM((2, D_IN), jnp.float32),
        pltpu.VMEM((2, K, D_IN), jnp.float32),
        pltpu.VMEM((NG * L,), jnp.float32),
        pltpu.VMEM((NG * L,), jnp.int32),
        pltpu.VMEM((NH * L,), jnp.float32),
        pltpu.VMEM((NH * L,), jnp.int32),
        pltpu.VMEM((L,), jnp.float32),
        pltpu.VMEM((L,), jnp.int32),
        pltpu.VMEM((2, K), jnp.int32),
        pltpu.VMEM((tpw * K,), jnp.float32),
        pltpu.VMEM((tpw * K,), jnp.int32),
        pltpu.VMEM((D_IN,), jnp.float32),
        pltpu.VMEM((D_IN,), jnp.float32),
        pltpu.VMEM((L,), jnp.float32),
        pltpu.SemaphoreType.DMA,
        pltpu.SemaphoreType.DMA,
        pltpu.SemaphoreType.DMA,
        pltpu.SemaphoreType.DMA,
    ]
    return pl.kernel(
        _make_sc_body(tpw), out_type=out_type, mesh=mesh,
        scratch_types=scratch,
    )(pre_acts, x, W_dec, b_dec)


def kernel(x, W_enc, b_enc, W_dec, b_dec):
    outs = []
    off = 0
    for n_c in CHUNK_SIZES:
        xc = x[off:off + n_c]
        off += n_c
        pre_c = _encode(xc, W_enc, b_enc, b_dec)
        outs.append(_sc_decode(pre_c, xc, W_dec, b_dec))
    nck = len(CHUNK_SIZES)
    sae_out = jnp.concatenate([o[0] for o in outs], axis=0)
    ta = jnp.concatenate([o[1] for o in outs])
    ti = jnp.concatenate([o[2] for o in outs])
    pcol = jnp.stack([o[3] for o in outs])
    pse = jnp.stack([o[4] for o in outs])
    top_acts = ta.reshape(N_TOK, K)
    top_indices = ti.reshape(N_TOK, K)
    colsum = pcol.reshape(nck * NW, D_IN).sum(axis=0)
    pse2 = pse.reshape(nck * NW, 2, L)
    sq_tot = pse2[:, 0, :].sum()
    e2_tot = pse2[:, 1, :].sum()
    total_variance = sq_tot - jnp.sum(colsum * colsum) / N_TOK
    fvu = e2_tot / total_variance
    auxk_loss = jnp.array(0.0, dtype=jnp.float32)
    multi_topk_fvu = jnp.array(0.0, dtype=jnp.float32)
    return (sae_out, top_acts, top_indices, fvu, auxk_loss, multi_topk_fvu)


# speedup vs baseline: 1.1179x; 1.0047x over previous
import functools

import jax
import jax.numpy as jnp
from jax import lax
from jax.experimental import pallas as pl
from jax.experimental.pallas import tpu as pltpu
from jax.experimental.pallas import tpu_sc as plsc

D_IN = 768
NUM_LATENTS = 24576
K = 32
N_TOK = 2048

TM = 512
TN = 2048
CHUNK_SIZES = (512, 512, 512, 512)


def _encode_body(x_ref, w_ref, benc_ref, bdec_ref, out_ref):
    xc = x_ref[...] - bdec_ref[...]
    acts = lax.dot_general(
        xc, w_ref[...], (((1,), (1,)), ((), ())),
        preferred_element_type=jnp.float32,
    )
    out_ref[...] = jnp.maximum(acts + benc_ref[...], 0.0)


def _encode(x, W_enc, b_enc, b_dec):
    n = x.shape[0]
    grid = (n // TM, NUM_LATENTS // TN)
    return pl.pallas_call(
        _encode_body,
        grid=grid,
        in_specs=[
            pl.BlockSpec((TM, D_IN), lambda i, j: (i, 0)),
            pl.BlockSpec((TN, D_IN), lambda i, j: (j, 0)),
            pl.BlockSpec((1, TN), lambda i, j: (0, j)),
            pl.BlockSpec((1, D_IN), lambda i, j: (0, 0)),
        ],
        out_specs=pl.BlockSpec((TM, TN), lambda i, j: (i, j)),
        out_shape=jax.ShapeDtypeStruct((n, NUM_LATENTS), jnp.float32),
        compiler_params=pltpu.CompilerParams(
            dimension_semantics=("parallel", "parallel"),
        ),
    )(x, W_enc, b_enc.reshape(1, NUM_LATENTS), b_dec.reshape(1, D_IN))


NC = 2
NS = 16
NW = NC * NS
TPW = N_TOK // NW
L = 16
NVR = NUM_LATENTS // L
NG = NVR // L
NH = NG // L
DCH = D_IN // L
NEG = -1.0


def _make_sc_body(tpw):
  def _sc_body(pre_hbm, x_hbm, wdec_hbm, bdec_hbm,
               sae_hbm, ta_hbm, ti_hbm, pcol_hbm, pse_hbm,
               row2, x4, out2, wrows2, l1v, l1i, l2v, l2i, rv, ri, tik2,
               tva, tia, bdec_v, colacc, s16f,
               sem_row, sem_x, sem_out, sem_g):
    wid = lax.axis_index("s") * NC + lax.axis_index("c")
    w_base = wid * tpw
    iota = lax.iota(jnp.int32, L)
    zf = jnp.zeros((L,), jnp.float32)
    zi = jnp.zeros((L,), jnp.int32)
    negv = jnp.full((L,), NEG, jnp.float32)
    jconsts = [jnp.full((L,), j, jnp.int32) for j in range(L)]

    pltpu.sync_copy(bdec_hbm, bdec_v)
    pltpu.async_copy(pre_hbm.at[w_base], row2.at[0], sem_row)
    pltpu.async_copy(x_hbm.at[w_base], x4.at[pl.ds(0, D_IN)], sem_x)
    for d in range(DCH):
        colacc[pl.ds(d * L, L)] = zf

    def _tree(pairs):
        while len(pairs) > 1:
            nxt = []
            for a in range(0, len(pairs) - 1, 2):
                (av, ai), (bv, bi) = pairs[a], pairs[a + 1]
                m = bv > av
                nxt.append((jnp.where(m, bv, av), jnp.where(m, bi, ai)))
            if len(pairs) % 2:
                nxt.append(pairs[-1])
            pairs = nxt
        return pairs[0]

    def fold_l1(row_ref, g_el):
        base = g_el * (L * L)
        pairs = [(row_ref[pl.ds(base + j * L, L)], jconsts[j])
                 for j in range(L)]
        acc_v, acc_j = _tree(pairs)
        l1v[pl.ds(g_el * L, L)] = acc_v
        l1i[pl.ds(g_el * L, L)] = acc_j

    def fold_l2(h_el):
        base = h_el * (L * L)
        pairs = [(l1v[pl.ds(base + g * L, L)],
                  l1i[pl.ds(base + g * L, L)] + jconsts[g] * L)
                 for g in range(L)]
        acc_v, acc_i = _tree(pairs)
        l2v[pl.ds(h_el * L, L)] = acc_v
        l2i[pl.ds(h_el * L, L)] = acc_i

    def fold_r():
        pairs = [(l2v[pl.ds(h * L, L)],
                  l2i[pl.ds(h * L, L)] + jconsts[h] * (L * L))
                 for h in range(NH)]
        return _tree(pairs)

    def decode_token(q, phq, pv0, pv1, sqv, e2v):
        q_glob = w_base + q
        pltpu.make_async_copy(
            wdec_hbm.at[tik2.at[phq]], wrows2.at[phq], sem_g).wait()
        avs = ([jnp.broadcast_to(pv0[k], (L,)) for k in range(L)]
               + [jnp.broadcast_to(pv1[k], (L,)) for k in range(L)])
        xbase = (q & 3) * D_IN

        def dchunk(d, car2):
            sqv_, e2v_ = car2
            sl = pl.ds(d * L, L)
            terms = [avs[k] * wrows2[phq, k, sl] for k in range(K)]
            terms.append(bdec_v[sl])
            while len(terms) > 1:
                terms = ([terms[a] + terms[a + 1]
                          for a in range(0, len(terms) - 1, 2)]
                         + ([terms[-1]] if len(terms) % 2 else []))
            acc = terms[0]
            xv = x4[pl.ds(xbase + d * L, L)]
            out2[phq, sl] = acc
            e = acc - xv
            colacc[sl] = colacc[sl] + xv
            return sqv_ + xv * xv, e2v_ + e * e

        sqv, e2v = lax.fori_loop(0, DCH, dchunk, (sqv, e2v))
        pltpu.async_copy(out2.at[phq], sae_hbm.at[q_glob], sem_out)
        return sqv, e2v

    def token_step(tt, ph, carry, do_decode=True):
        sqv, e2v, pv0, pv1 = carry
        tt = jnp.int32(tt)
        t_glob = w_base + tt
        pltpu.make_async_copy(pre_hbm.at[t_glob], row2.at[ph], sem_row).wait()
        pltpu.make_async_copy(
            x_hbm.at[t_glob],
            x4.at[pl.ds((tt & 3) * D_IN, D_IN)], sem_x).wait()

        @pl.when(tt + 1 < tpw)
        def _prefetch():
            pltpu.async_copy(pre_hbm.at[t_glob + 1], row2.at[1 - ph], sem_row)
            pltpu.async_copy(
                x_hbm.at[t_glob + 1],
                x4.at[pl.ds(((tt + 1) & 3) * D_IN, D_IN)], sem_x)

        row_ref = row2.at[ph]

        def build1(g, _):
            fold_l1(row_ref, g)
            return 0
        lax.fori_loop(0, NG, build1, 0)
        for h in range(NH):
            fold_l2(h)
        rv0, ri0 = fold_r()

        def extract(i, car):
            ov0, oi0, ov1, oi1, rvr, rir = car
            bv = rvr
            bg = rir * L + iota
            for dd in (8, 4, 2, 1):
                perm = iota ^ dd
                o_v = jnp.take(bv, perm)
                o_g = jnp.take(bg, perm)
                better = jnp.logical_or(
                    o_v > bv, jnp.logical_and(o_v == bv, o_g < bg))
                bv = jnp.where(better, o_v, bv)
                bg = jnp.where(better, o_g, bg)
            m0 = iota == jnp.broadcast_to(i, (L,))
            m1 = iota == jnp.broadcast_to(i - L, (L,))
            ov0 = jnp.where(m0, bv, ov0)
            oi0 = jnp.where(m0, bg, oi0)
            ov1 = jnp.where(m1, bv, ov1)
            oi1 = jnp.where(m1, bg, oi1)
            gidx = bg[0]
            vj = gidx >> 4
            lane = gidx & (L - 1)
            off = vj * L
            vcur = row_ref[pl.ds(off, L)]
            row_ref[pl.ds(off, L)] = jnp.where(
                iota == jnp.broadcast_to(lane, (L,)), negv, vcur)
            fold_l1(row_ref, vj >> 4)
            fold_l2(vj >> 8)
            rvr, rir = fold_r()
            return ov0, oi0, ov1, oi1, rvr, rir

        ov0, oi0, ov1, oi1, _, _ = lax.fori_loop(
            0, K, extract, (negv, zi, negv, zi, rv0, ri0))

        off_t = tt * K
        tik2[ph, pl.ds(0, L)] = oi0
        tik2[ph, pl.ds(L, L)] = oi1
        tva[pl.ds(off_t, L)] = ov0
        tva[pl.ds(off_t + L, L)] = ov1
        tia[pl.ds(off_t, L)] = oi0
        tia[pl.ds(off_t + L, L)] = oi1

        pltpu.async_copy(wdec_hbm.at[tik2.at[ph]], wrows2.at[ph], sem_g)

        if do_decode:
            @pl.when(tt >= 3)
            def _drain_out():
                pltpu.make_async_copy(
                    out2.at[1 - ph], sae_hbm.at[t_glob - 3], sem_out).wait()
            sqv, e2v = decode_token(tt - 1, 1 - ph, pv0, pv1, sqv, e2v)
        return sqv, e2v, ov0, ov1

    carry = token_step(0, 0, (zf, zf, zf, zf), do_decode=False)

    def pair_step(p, carry):
        carry = token_step(2 * p + 1, 1, carry)
        carry = token_step(2 * p + 2, 0, carry)
        return carry

    carry = lax.fori_loop(0, (tpw - 2) // 2, pair_step, carry)
    sqv, e2v, pv0, pv1 = token_step(tpw - 1, 1, carry)

    phl = (tpw - 1) & 1
    pltpu.make_async_copy(
        out2.at[phl], sae_hbm.at[w_base + tpw - 3], sem_out).wait()
    sqv, e2v = decode_token(tpw - 1, phl, pv0, pv1, sqv, e2v)
    pltpu.make_async_copy(
        out2.at[1 - phl], sae_hbm.at[w_base + tpw - 2], sem_out).wait()
    pltpu.make_async_copy(
        out2.at[phl], sae_hbm.at[w_base + tpw - 1], sem_out).wait()

    pltpu.sync_copy(tva, ta_hbm.at[pl.ds(w_base * K, tpw * K)])
    pltpu.sync_copy(tia, ti_hbm.at[pl.ds(w_base * K, tpw * K)])

    s16f[...] = sqv
    pltpu.sync_copy(s16f, pse_hbm.at[pl.ds(wid * 2 * L, L)])
    s16f[...] = e2v
    pltpu.sync_copy(s16f, pse_hbm.at[pl.ds(wid * 2 * L + L, L)])
    pltpu.sync_copy(colacc, pcol_hbm.at[pl.ds(wid * D_IN, D_IN)])

  return _sc_body


def _sc_decode(pre_acts, x, W_dec, b_dec):
    n = pre_acts.shape[0]
    tpw = n // NW
    mesh = plsc.VectorSubcoreMesh(
        core_axis_name="c", subcore_axis_name="s",
        num_cores=NC, num_subcores=NS)
    out_type = (
        jax.ShapeDtypeStruct((n, D_IN), jnp.float32),
        jax.ShapeDtypeStruct((n * K,), jnp.float32),
        jax.ShapeDtypeStruct((n * K,), jnp.int32),
        jax.ShapeDtypeStruct((NW * D_IN,), jnp.float32),
        jax.ShapeDtypeStruct((NW * 2 * L,), jnp.float32),
    )
    scratch = [
        pltpu.VMEM((2, NUM_LATENTS), jnp.float32),
        pltpu.VMEM((4 * D_IN,), jnp.float32),
        pltpu.VMEM((2, D_IN), jnp.float32),
        pltpu.VMEM((2, K, D_IN), jnp.float32),
        pltpu.VMEM((NG * L,), jnp.float32),
        pltpu.VMEM((NG * L,), jnp.int32),
        pltpu.VMEM((NH * L,), jnp.float32),
        pltpu.VMEM((NH * L,), jnp.int32),
        pltpu.VMEM((L,), jnp.float32),
        pltpu.VMEM((L,), jnp.int32),
        pltpu.VMEM((2, K), jnp.int32),
        pltpu.VMEM((tpw * K,), jnp.float32),
        pltpu.VMEM((tpw * K,), jnp.int32),
        pltpu.VMEM((D_IN,), jnp.float32),
        pltpu.VMEM((D_IN,), jnp.float32),
        pltpu.VMEM((L,), jnp.float32),
        pltpu.SemaphoreType.DMA,
        pltpu.SemaphoreType.DMA,
        pltpu.SemaphoreType.DMA,
        pltpu.SemaphoreType.DMA,
    ]
    return pl.kernel(
        _make_sc_body(tpw), out_type=out_type, mesh=mesh,
        scratch_types=scratch,
    )(pre_acts, x, W_dec, b_dec)


def kernel(x, W_enc, b_enc, W_dec, b_dec):
    outs = []
    off = 0
    for n_c in CHUNK_SIZES:
        xc = x[off:off + n_c]
        off += n_c
        pre_c = _encode(xc, W_enc, b_enc, b_dec)
        outs.append(_sc_decode(pre_c, xc, W_dec, b_dec))
    nck = len(CHUNK_SIZES)
    sae_out = jnp.concatenate([o[0] for o in outs], axis=0)
    ta = jnp.concatenate([o[1] for o in outs])
    ti = jnp.concatenate([o[2] for o in outs])
    pcol = jnp.stack([o[3] for o in outs])
    pse = jnp.stack([o[4] for o in outs])
    top_acts = ta.reshape(N_TOK, K)
    top_indices = ti.reshape(N_TOK, K)
    colsum = pcol.reshape(nck * NW, D_IN).sum(axis=0)
    pse2 = pse.reshape(nck * NW, 2, L)
    sq_tot = pse2[:, 0, :].sum()
    e2_tot = pse2[:, 1, :].sum()
    total_variance = sq_tot - jnp.sum(colsum * colsum) / N_TOK
    fvu = e2_tot / total_variance
    auxk_loss = jnp.array(0.0, dtype=jnp.float32)
    multi_topk_fvu = jnp.array(0.0, dtype=jnp.float32)
    return (sae_out, top_acts, top_indices, fvu, auxk_loss, multi_topk_fvu)
